# baseline (speedup 1.0000x reference)
import functools

import jax
import jax.numpy as jnp
from jax import lax
from jax.experimental import pallas as pl
from jax.experimental.pallas import tpu as pltpu
from jax.experimental.pallas import tpu_sc as plsc

NUM_EMB = 1_000_000
DIM = 32
N_IDS = 819_200
N_BAGS = 16_384

NC = 2
NS = 16
NW = NC * NS
BPW = N_BAGS // NW
SUB = 128
NSUB = 8
CHUNK = SUB * NSUB
OFFPAD = BPW + 32

_mesh = plsc.VectorSubcoreMesh(core_axis_name="c", subcore_axis_name="s")

W = 768
NBLK = NUM_EMB // W
TAILC = NUM_EMB - NBLK * W
BLKPW = -(-NBLK // NW)


@functools.partial(
    pl.kernel,
    out_type=jax.ShapeDtypeStruct((NUM_EMB * DIM,), jnp.float32),
    mesh=_mesh,
    scratch_types=[
        pltpu.VMEM((2 * DIM, W), jnp.float32),
        pltpu.VMEM((2 * W * DIM,), jnp.float32),
        pltpu.VMEM((DIM, TAILC), jnp.float32),
        pltpu.VMEM((TAILC * DIM,), jnp.float32),
        pltpu.SemaphoreType.DMA,
        pltpu.SemaphoreType.DMA,
        pltpu.SemaphoreType.DMA,
        pltpu.SemaphoreType.DMA,
    ],
    compiler_params=pltpu.CompilerParams(
        needs_layout_passes=False, use_tc_tiling_on_sc=True
    ),
)
def _transpose(tableT, tlin, tin, tout, tint, toutt, si0, si1, so0, so1):
    wid = lax.axis_index("s") * NC + lax.axis_index("c")
    j0 = wid * BLKPW
    j1 = jnp.minimum(j0 + BLKPW, NBLK)
    lanes = lax.iota(jnp.int32, 16)

    def fire_in(j, d0, sem):
        c = pl.multiple_of(j * W, W)
        pltpu.async_copy(
            tableT.at[:, pl.ds(c, W)], tin.at[pl.ds(d0, DIM)], sem
        )

    def drain_in(d0, sem):
        pltpu.make_async_copy(
            tableT.at[:, pl.ds(0, W)], tin.at[pl.ds(d0, DIM)], sem
        ).wait()

    L32 = lanes * DIM

    def compute(src, d0, dst, dst_base, ids_n):
        pairs = [(s, h) for s in range(16) for h in range(2)]

        def grp(q, carry):
            cc = q * 16 + lanes
            qb = dst_base + q * (16 * DIM)
            for g in range(0, 32, 8):
                batch = []
                for s, h in pairs[g:g + 8]:
                    rot = (lanes + s) & 15
                    v = plsc.load_gather(src, [rot + (d0 + 16 * h), cc])
                    batch.append((rot, h, v))
                for rot, h, v in batch:
                    plsc.store_scatter(
                        dst, [L32 + rot + (qb + 16 * h)], v
                    )
            return carry

        lax.fori_loop(0, ids_n // 16, grp, 0)

    def fire_out(j, ob, sem):
        pltpu.async_copy(
            tout.at[pl.ds(ob, W * DIM)],
            tlin.at[pl.ds(j * (W * DIM), W * DIM)],
            sem,
        )

    def drain_out(ob, sem):
        pltpu.make_async_copy(
            tout.at[pl.ds(ob, W * DIM)],
            tlin.at[pl.ds(0, W * DIM)],
            sem,
        ).wait()

    @pl.when(j0 < j1)
    def _():
        fire_in(j0, 0, si0)

    def cond(c):
        return c[0] < j1

    def body(c):
        (j,) = c

        @pl.when(j + 1 < j1)
        def _():
            fire_in(j + 1, DIM, si1)

        drain_in(0, si0)

        @pl.when(j - 2 >= j0)
        def _():
            drain_out(0, so0)

        compute(tin, 0, tout, 0, W)
        fire_out(j, 0, so0)

        @pl.when(j + 2 < j1)
        def _():
            fire_in(j + 2, 0, si0)

        @pl.when(j + 1 < j1)
        def _():
            drain_in(DIM, si1)

            @pl.when(j - 1 >= j0)
            def _():
                drain_out(W * DIM, so1)

            compute(tin, DIM, tout, W * DIM, W)
            fire_out(j + 1, W * DIM, so1)

        return (j + 2,)

    lax.while_loop(cond, body, (j0,))

    @pl.when(j1 > j0)
    def _():
        drain_out(0, so0)

    @pl.when(j1 > j0 + 1)
    def _():
        drain_out(W * DIM, so1)

    @pl.when(wid == NW - 1)
    def _():
        pltpu.sync_copy(tableT.at[:, pl.ds(NBLK * W, TAILC)], tint)
        compute(tint, 0, toutt, 0, TAILC)
        pltpu.sync_copy(
            toutt, tlin.at[pl.ds(NBLK * W * DIM, TAILC * DIM)]
        )


@functools.partial(
    pl.kernel,
    out_type=jax.ShapeDtypeStruct((N_BAGS * DIM,), jnp.float32),
    mesh=_mesh,
    scratch_types=[
        pltpu.VMEM((2 * NSUB, SUB), jnp.int32),
        pltpu.VMEM((2 * CHUNK, DIM), jnp.float32),
        pltpu.VMEM((BPW * DIM,), jnp.float32),
        pltpu.VMEM((OFFPAD,), jnp.int32),
        pltpu.SemaphoreType.DMA,
        pltpu.SemaphoreType.DMA,
    ],
    compiler_params=pltpu.CompilerParams(
        needs_layout_passes=False, use_tc_tiling_on_sc=False
    ),
)
def _emb_bag(ids_hbm, off_hbm, table_hbm, out_hbm, idsv, rows, accv, offv,
             sem_a, sem_b):
    wid = lax.axis_index("s") * NC + lax.axis_index("c")
    pltpu.sync_copy(off_hbm.at[pl.ds(wid * BPW, OFFPAD)], offv)

    zero16 = jnp.zeros((16,), jnp.float32)

    def zero_body(i, carry):
        accv[pl.ds(i * 16, 16)] = zero16
        return carry

    lax.fori_loop(0, BPW * DIM // 16, zero_body, 0)

    lanes = lax.iota(jnp.int32, 16)

    def off_at(i):
        al = (i // 8) * 8
        vec = offv[pl.ds(al, 16)]
        return jnp.sum(jnp.where(lanes == i - al, vec, 0))

    start = off_at(0)
    end = off_at(BPW)
    base = (start // CHUNK) * CHUNK

    def fire(cs, ib, rb, sem):
        row0 = pl.multiple_of(cs // SUB, 8)
        pltpu.sync_copy(ids_hbm.at[pl.ds(row0, NSUB)], idsv.at[pl.ds(ib, NSUB)])
        for j in range(NSUB):
            pltpu.async_copy(
                table_hbm.at[idsv.at[ib + j]],
                rows.at[pl.ds(rb + j * SUB, SUB)],
                sem,
            )

    def drain(ib, rb, sem):
        for j in range(NSUB):
            pltpu.make_async_copy(
                table_hbm.at[idsv.at[ib + j]],
                rows.at[pl.ds(rb + j * SUB, SUB)],
                sem,
            ).wait()

    def accum(cs, rb, b, r, a0, a1):
        ce = jnp.minimum(cs + CHUNK, end)

        def span_cond(s):
            _, r, _, _ = s
            return r < ce

        def span_body(s):
            b, r, a0, a1 = s
            bag_end = off_at(b + 1)
            e = jnp.minimum(bag_end, ce)
            nfull = (e - r) // 8

            def oct_body(q, accs):
                loc = rb + (r - cs) + q * 8
                acc = list(accs)
                for j in range(8):
                    acc[2 * j] = acc[2 * j] + rows[loc + j, pl.ds(0, 16)]
                    acc[2 * j + 1] = acc[2 * j + 1] + rows[loc + j, pl.ds(16, 16)]
                return tuple(acc)

            z = (zero16,) * 14
            accs = lax.fori_loop(0, nfull, oct_body, (a0, a1) + z)
            a0 = (accs[0] + accs[2]) + (accs[4] + accs[6]) + (
                (accs[8] + accs[10]) + (accs[12] + accs[14]))
            a1 = (accs[1] + accs[3]) + (accs[5] + accs[7]) + (
                (accs[9] + accs[11]) + (accs[13] + accs[15]))

            def tail_body(rr, accs):
                t0, t1 = accs
                loc = rb + (rr - cs)
                t0 = t0 + rows[loc, pl.ds(0, 16)]
                t1 = t1 + rows[loc, pl.ds(16, 16)]
                return (t0, t1)

            a0, a1 = lax.fori_loop(r + nfull * 8, e, tail_body, (a0, a1))
            done = e == bag_end

            @pl.when(done)
            def _():
                accv[pl.ds(b * DIM, 16)] = a0
                accv[pl.ds(b * DIM + 16, 16)] = a1

            b = b + done.astype(jnp.int32)
            a0 = jnp.where(done, zero16, a0)
            a1 = jnp.where(done, zero16, a1)
            return (b, e, a0, a1)

        return lax.while_loop(span_cond, span_body, (b, r, a0, a1))

    IB_A, RB_A = 0, 0
    IB_B, RB_B = NSUB, CHUNK

    @pl.when(start < end)
    def _():
        fire(base, IB_A, RB_A, sem_a)

    def chunk_cond(c):
        _, _, r, _, _ = c
        return r < end

    def chunk_body(c):
        cs0, b, r, a0, a1 = c
        cs1 = cs0 + CHUNK
        cs2 = cs1 + CHUNK

        @pl.when(cs1 < end)
        def _():
            fire(cs1, IB_B, RB_B, sem_b)

        drain(IB_A, RB_A, sem_a)
        b, r, a0, a1 = accum(cs0, RB_A, b, r, a0, a1)

        @pl.when(cs2 < end)
        def _():
            fire(cs2, IB_A, RB_A, sem_a)

        @pl.when(cs1 < end)
        def _():
            drain(IB_B, RB_B, sem_b)

        b, r, a0, a1 = accum(cs1, RB_B, b, r, a0, a1)
        return (cs2, b, r, a0, a1)

    init = (base, jnp.int32(0), start, zero16, zero16)
    lax.while_loop(chunk_cond, chunk_body, init)

    pltpu.sync_copy(accv, out_hbm.at[pl.ds(wid * BPW * DIM, BPW * DIM)])


def kernel(ids, offset, table):
    pad_ids = (jnp.arange(2 * CHUNK, dtype=jnp.int32) * 997) % NUM_EMB
    ids2d = jnp.concatenate([ids, pad_ids]).reshape(-1, SUB)
    off_pad = jnp.concatenate(
        [offset, jnp.full((OFFPAD - BPW,), N_IDS, jnp.int32)]
    )
    table_lin = _transpose(table.T).reshape(NUM_EMB, DIM)
    out = _emb_bag(ids2d, off_pad, table_lin)
    return out.reshape(N_BAGS, DIM)


# speedup vs baseline: 1.0282x; 1.0282x over previous
import functools

import jax
import jax.numpy as jnp
from jax import lax
from jax.experimental import pallas as pl
from jax.experimental.pallas import tpu as pltpu
from jax.experimental.pallas import tpu_sc as plsc

NUM_EMB = 1_000_000
DIM = 32
N_IDS = 819_200
N_BAGS = 16_384

NC = 2
NS = 16
NW = NC * NS
BPW = N_BAGS // NW
SUB = 128
NSUB = 8
CHUNK = SUB * NSUB
OFFPAD = BPW + 32

_mesh = plsc.VectorSubcoreMesh(core_axis_name="c", subcore_axis_name="s")

W = 768
NBLK = NUM_EMB // W
TAILC = NUM_EMB - NBLK * W
BLKPW = -(-NBLK // NW)


@functools.partial(
    pl.kernel,
    out_type=jax.ShapeDtypeStruct((NUM_EMB * DIM,), jnp.float32),
    mesh=_mesh,
    scratch_types=[
        pltpu.VMEM((2 * DIM, W), jnp.float32),
        pltpu.VMEM((2 * W * DIM,), jnp.float32),
        pltpu.VMEM((DIM, TAILC), jnp.float32),
        pltpu.VMEM((TAILC * DIM,), jnp.float32),
        pltpu.SemaphoreType.DMA,
        pltpu.SemaphoreType.DMA,
        pltpu.SemaphoreType.DMA,
        pltpu.SemaphoreType.DMA,
    ],
    compiler_params=pltpu.CompilerParams(
        needs_layout_passes=False, use_tc_tiling_on_sc=True
    ),
)
def _transpose(tableT, tlin, tin, tout, tint, toutt, si0, si1, so0, so1):
    wid = lax.axis_index("s") * NC + lax.axis_index("c")
    j0 = wid * BLKPW
    j1 = jnp.minimum(j0 + BLKPW, NBLK)
    lanes = lax.iota(jnp.int32, 16)

    def fire_in(j, d0, sem):
        c = pl.multiple_of(j * W, W)
        pltpu.async_copy(
            tableT.at[:, pl.ds(c, W)], tin.at[pl.ds(d0, DIM)], sem
        )

    def drain_in(d0, sem):
        pltpu.make_async_copy(
            tableT.at[:, pl.ds(0, W)], tin.at[pl.ds(d0, DIM)], sem
        ).wait()

    L32 = lanes * DIM

    def compute(src, d0, dst, dst_base, ids_n):
        pairs = [(s, h) for s in range(16) for h in range(2)]

        def grp(q, carry):
            cc = q * 16 + lanes
            qb = dst_base + q * (16 * DIM)
            for g in range(0, 32, 8):
                batch = []
                for s, h in pairs[g:g + 8]:
                    rot = (lanes + s) & 15
                    v = plsc.load_gather(src, [rot + (d0 + 16 * h), cc])
                    batch.append((rot, h, v))
                for rot, h, v in batch:
                    plsc.store_scatter(
                        dst, [L32 + rot + (qb + 16 * h)], v
                    )
            return carry

        lax.fori_loop(0, ids_n // 16, grp, 0)

    def fire_out(j, ob, sem):
        pltpu.async_copy(
            tout.at[pl.ds(ob, W * DIM)],
            tlin.at[pl.ds(j * (W * DIM), W * DIM)],
            sem,
        )

    def drain_out(ob, sem):
        pltpu.make_async_copy(
            tout.at[pl.ds(ob, W * DIM)],
            tlin.at[pl.ds(0, W * DIM)],
            sem,
        ).wait()

    @pl.when(j0 < j1)
    def _():
        fire_in(j0, 0, si0)

    def cond(c):
        return c[0] < j1

    def body(c):
        (j,) = c

        @pl.when(j + 1 < j1)
        def _():
            fire_in(j + 1, DIM, si1)

        drain_in(0, si0)

        @pl.when(j - 2 >= j0)
        def _():
            drain_out(0, so0)

        compute(tin, 0, tout, 0, W)
        fire_out(j, 0, so0)

        @pl.when(j + 2 < j1)
        def _():
            fire_in(j + 2, 0, si0)

        @pl.when(j + 1 < j1)
        def _():
            drain_in(DIM, si1)

            @pl.when(j - 1 >= j0)
            def _():
                drain_out(W * DIM, so1)

            compute(tin, DIM, tout, W * DIM, W)
            fire_out(j + 1, W * DIM, so1)

        return (j + 2,)

    lax.while_loop(cond, body, (j0,))

    @pl.when(j1 > j0)
    def _():
        drain_out(0, so0)

    @pl.when(j1 > j0 + 1)
    def _():
        drain_out(W * DIM, so1)

    @pl.when(wid == NW - 1)
    def _():
        pltpu.sync_copy(tableT.at[:, pl.ds(NBLK * W, TAILC)], tint)
        compute(tint, 0, toutt, 0, TAILC)
        pltpu.sync_copy(
            toutt, tlin.at[pl.ds(NBLK * W * DIM, TAILC * DIM)]
        )


@functools.partial(
    pl.kernel,
    out_type=jax.ShapeDtypeStruct((N_BAGS * DIM,), jnp.float32),
    mesh=_mesh,
    scratch_types=[
        pltpu.VMEM((2 * NSUB, SUB), jnp.int32),
        pltpu.VMEM((2 * CHUNK, DIM), jnp.float32),
        pltpu.VMEM((BPW * DIM,), jnp.float32),
        pltpu.VMEM((OFFPAD,), jnp.int32),
        pltpu.SemaphoreType.DMA,
        pltpu.SemaphoreType.DMA,
    ],
    compiler_params=pltpu.CompilerParams(
        needs_layout_passes=False, use_tc_tiling_on_sc=False
    ),
)
def _emb_bag(ids_hbm, off_hbm, table_hbm, out_hbm, idsv, rows, accv, offv,
             sem_a, sem_b):
    wid = lax.axis_index("s") * NC + lax.axis_index("c")
    pltpu.sync_copy(off_hbm.at[pl.ds(wid * BPW, OFFPAD)], offv)

    zero16 = jnp.zeros((16,), jnp.float32)
    lanes = lax.iota(jnp.int32, 16)

    def off_at(i):
        al = (i // 8) * 8
        vec = offv[pl.ds(al, 16)]
        return jnp.sum(jnp.where(lanes == i - al, vec, 0))

    start = off_at(0)
    end = off_at(BPW)
    base = (start // CHUNK) * CHUNK

    def fire(cs, ib, rb, sem):
        row0 = pl.multiple_of(cs // SUB, 8)
        pltpu.sync_copy(ids_hbm.at[pl.ds(row0, NSUB)], idsv.at[pl.ds(ib, NSUB)])
        for j in range(NSUB):
            pltpu.async_copy(
                table_hbm.at[idsv.at[ib + j]],
                rows.at[pl.ds(rb + j * SUB, SUB)],
                sem,
            )

    def drain(ib, rb, sem):
        for j in range(NSUB):
            pltpu.make_async_copy(
                table_hbm.at[idsv.at[ib + j]],
                rows.at[pl.ds(rb + j * SUB, SUB)],
                sem,
            ).wait()

    def accum(cs, rb, b, r, a0, a1):
        ce = jnp.minimum(cs + CHUNK, end)

        def span_cond(s):
            _, r, _, _ = s
            return r < ce

        def span_body(s):
            b, r, a0, a1 = s
            bag_end = off_at(b + 1)
            e = jnp.minimum(bag_end, ce)
            nfull = (e - r) // 4

            def quad_body(q, accs):
                loc = rb + (r - cs) + q * 4
                acc = list(accs)
                for j in range(4):
                    acc[2 * j] = acc[2 * j] + rows[loc + j, pl.ds(0, 16)]
                    acc[2 * j + 1] = acc[2 * j + 1] + rows[loc + j, pl.ds(16, 16)]
                return tuple(acc)

            z = (zero16,) * 6
            accs = lax.fori_loop(0, nfull, quad_body, (a0, a1) + z)
            a0 = accs[0] + accs[2] + accs[4] + accs[6]
            a1 = accs[1] + accs[3] + accs[5] + accs[7]

            def tail_body(rr, accs):
                t0, t1 = accs
                loc = rb + (rr - cs)
                t0 = t0 + rows[loc, pl.ds(0, 16)]
                t1 = t1 + rows[loc, pl.ds(16, 16)]
                return (t0, t1)

            a0, a1 = lax.fori_loop(r + nfull * 4, e, tail_body, (a0, a1))
            done = e == bag_end

            @pl.when(done)
            def _():
                accv[pl.ds(b * DIM, 16)] = a0
                accv[pl.ds(b * DIM + 16, 16)] = a1

            b = b + done.astype(jnp.int32)
            a0 = jnp.where(done, zero16, a0)
            a1 = jnp.where(done, zero16, a1)
            return (b, e, a0, a1)

        return lax.while_loop(span_cond, span_body, (b, r, a0, a1))

    IB_A, RB_A = 0, 0
    IB_B, RB_B = NSUB, CHUNK

    @pl.when(start < end)
    def _():
        fire(base, IB_A, RB_A, sem_a)

    def zero_body(i, carry):
        accv[pl.ds(i * 16, 16)] = zero16
        return carry

    lax.fori_loop(0, BPW * DIM // 16, zero_body, 0)

    def chunk_cond(c):
        _, _, r, _, _ = c
        return r < end

    def chunk_body(c):
        cs0, b, r, a0, a1 = c
        cs1 = cs0 + CHUNK
        cs2 = cs1 + CHUNK

        @pl.when(cs1 < end)
        def _():
            fire(cs1, IB_B, RB_B, sem_b)

        drain(IB_A, RB_A, sem_a)
        b, r, a0, a1 = accum(cs0, RB_A, b, r, a0, a1)

        @pl.when(cs2 < end)
        def _():
            fire(cs2, IB_A, RB_A, sem_a)

        @pl.when(cs1 < end)
        def _():
            drain(IB_B, RB_B, sem_b)

        b, r, a0, a1 = accum(cs1, RB_B, b, r, a0, a1)
        return (cs2, b, r, a0, a1)

    init = (base, jnp.int32(0), start, zero16, zero16)
    lax.while_loop(chunk_cond, chunk_body, init)

    pltpu.sync_copy(accv, out_hbm.at[pl.ds(wid * BPW * DIM, BPW * DIM)])


def kernel(ids, offset, table):
    pad_ids = (jnp.arange(2 * CHUNK, dtype=jnp.int32) * 997) % NUM_EMB
    ids2d = jnp.concatenate([ids, pad_ids]).reshape(-1, SUB)
    off_pad = jnp.concatenate(
        [offset, jnp.full((OFFPAD - BPW,), N_IDS, jnp.int32)]
    )
    table_lin = _transpose(table.T).reshape(NUM_EMB, DIM)
    out = _emb_bag(ids2d, off_pad, table_lin)
    return out.reshape(N_BAGS, DIM)
